# baseline (speedup 1.0000x reference)
import numpy as np
import jax
import jax.numpy as jnp
from jax.experimental import pallas as pl
from jax.experimental.pallas import tpu as pltpu

_RATIOS = np.array([0.5, 1.0, 2.0])
_SCALES = np.array([2 ** 0, 2 ** (1.0 / 3.0), 2 ** (2.0 / 3.0)])
_TOTAL = 48960
_CHUNK = 1152


def _anchor_sizes(box_size):
    anchors = box_size * np.tile(_SCALES, (2, len(_RATIOS))).T
    areas = anchors[:, 0] * anchors[:, 1]
    anchors[:, 0] = np.sqrt(areas * np.repeat(_RATIOS, len(_SCALES)))
    anchors[:, 1] = anchors[:, 0] / np.repeat(_RATIOS, len(_SCALES))
    return anchors.astype(np.float32)


_LEVELS = []
_lane_base = 0
for _w, _stride, _size in ((64, 8, 32), (32, 16, 64), (16, 32, 128), (8, 64, 256)):
    _n = _w * _w * 9
    _LEVELS.append((_w, float(_stride), _lane_base, _n,
                    [float(v) for v in _anchor_sizes(_size)[:, 0]]))
    _lane_base += _n

_C9 = float(np.float32(1.0 / 9.0))
_C3 = float(np.float32(1.0 / 3.0))


_GROUPS = ((0, 4608), (4608, 4608), (9216, 9216), (18432, 9216),
           (27648, 9216), (36864, 9216), (46080, 2880))


def _anchor_kernel(out1_ref, out2_ref, s1, s2, sem):
    f32, i32 = jnp.float32, jnp.int32

    def _start_group(g):
        lo, width = _GROUPS[g]
        for idx, (s, o) in enumerate(((s1, out1_ref), (s2, out2_ref))):
            pltpu.make_async_copy(s.at[:, pl.ds(lo, width)],
                                  o.at[:, pl.ds(lo, width)],
                                  sem.at[2 * g + idx]).start()

    done = 0
    next_group = 0
    for w, stride, base, n, wtab in _LEVELS:
        width = min(n, _CHUNK)
        shape = (4, width)
        lp = jax.lax.broadcasted_iota(i32, shape, 1).astype(f32)
        qf = jnp.floor((lp + 0.5) * _C9)
        af = lp - 9.0 * qf
        yq = jnp.floor(qf * (1.0 / w))
        xf = qf - yq * w
        cx = xf * stride + 0.5 * stride
        cy0 = yq * stride + 0.5 * stride
        wt = jnp.full(shape, wtab[8], f32)
        for k in range(7, -1, -1):
            wt = jnp.where(af == float(k), wtab[k], wt)
        tf = jnp.floor((af + 0.5) * _C3)
        pw = jnp.where(tf == 0.0, 2.0, jnp.where(tf == 1.0, 1.0, 0.5))
        ht = wt * pw
        j = jax.lax.broadcasted_iota(i32, shape, 0)
        even = (j & 1) == 0
        low = j < 2
        wh = jnp.where(even, wt, ht)
        habs = jnp.where(even, wt * 0.5, ht * 0.5)
        hsgn = jnp.where(low, -habs, habs)
        dy_step = stride * (width // 9) / w
        for i in range(n // width):
            cyv = cy0 + dy_step * i
            bc = jnp.where(even, cx, cyv)
            s1[:, pl.ds(base + i * width, width)] = jnp.where(low, bc, wh)
            s2[:, pl.ds(base + i * width, width)] = bc + hsgn
            done += width
            while next_group < len(_GROUPS) and \
                    done >= _GROUPS[next_group][0] + _GROUPS[next_group][1]:
                _start_group(next_group)
                next_group += 1
    for g in range(len(_GROUPS)):
        lo, width = _GROUPS[g]
        for idx, (s, o) in enumerate(((s1, out1_ref), (s2, out2_ref))):
            pltpu.make_async_copy(s.at[:, pl.ds(lo, width)],
                                  o.at[:, pl.ds(lo, width)],
                                  sem.at[2 * g + idx]).wait()


def kernel(feat0, feat1, feat2, feat3):
    del feat0, feat1, feat2, feat3
    o1t, o2t = pl.pallas_call(
        _anchor_kernel,
        out_shape=(jax.ShapeDtypeStruct((4, _TOTAL), jnp.float32),
                   jax.ShapeDtypeStruct((4, _TOTAL), jnp.float32)),
        out_specs=(pl.BlockSpec(memory_space=pl.ANY),
                   pl.BlockSpec(memory_space=pl.ANY)),
        scratch_shapes=[pltpu.VMEM((4, _TOTAL), jnp.float32),
                        pltpu.VMEM((4, _TOTAL), jnp.float32),
                        pltpu.SemaphoreType.DMA((14,))],
    )()
    return jnp.transpose(o1t), jnp.transpose(o2t)
